# baseline (speedup 1.0000x reference)
import functools

import jax
import jax.numpy as jnp
from jax import lax
from jax.experimental import pallas as pl
from jax.experimental.pallas import tpu as pltpu
from jax.experimental.pallas import tpu_sc as plsc

N = 10000
E = 320000
F = 128
G = 128
NPAD = 10240
EPS = 1e-5
NW = 32
EPW = E // NW
CH = 125
CHW = 128
NCH = EPW // CH
TRASH = NPAD - 1
CHP = 80
R = 1000
NB = N // R
f32 = jnp.float32

_mesh = plsc.VectorSubcoreMesh(core_axis_name="c", subcore_axis_name="s")


@functools.partial(
    pl.kernel, mesh=_mesh,
    out_type=jax.ShapeDtypeStruct((2, NPAD, 16), f32),
    scratch_types=[
        pltpu.VMEM((NCH, CHW), jnp.int32),
        pltpu.VMEM((CHW, 16), f32),
        pltpu.VMEM_SHARED((NPAD, 16), f32),
    ])
def _sc_degree(dst_hbm, ones_hbm, zeros_hbm, out_hbm, dst_v, ones_v, acc_sh):
    c = lax.axis_index("c")
    s = lax.axis_index("s")
    wid = c * 16 + s
    rpt = NPAD // 16
    pltpu.sync_copy(zeros_hbm.at[pl.ds(s * rpt, rpt)],
                    acc_sh.at[pl.ds(s * rpt, rpt)])
    pltpu.sync_copy(dst_hbm.at[wid], dst_v)
    pltpu.sync_copy(ones_hbm, ones_v)
    plsc.subcore_barrier()

    def body(j, _):
        pltpu.sync_copy(ones_v, acc_sh.at[dst_v.at[j]], add=True)
        return 0
    lax.fori_loop(0, NCH, body, 0)

    plsc.subcore_barrier()
    pltpu.sync_copy(acc_sh.at[pl.ds(s * rpt, rpt)],
                    out_hbm.at[c].at[pl.ds(s * rpt, rpt)])


@functools.partial(
    pl.kernel, mesh=_mesh,
    out_type=jax.ShapeDtypeStruct((2, NPAD, F), f32),
    scratch_types=[
        pltpu.VMEM((NCH, CHW), jnp.int32),
        pltpu.VMEM((1, CHW), jnp.int32),
        pltpu.VMEM((1, CHW), jnp.int32),
        pltpu.VMEM((CHW, F), f32),
        pltpu.VMEM((CHW, F), f32),
        pltpu.VMEM_SHARED((NPAD, F), f32),
        pltpu.SemaphoreType.DMA,
        pltpu.SemaphoreType.DMA,
    ])
def _sc_prop(u_hbm, src_hbm, dst_hbm, zeros_hbm, out_hbm,
             src_v, didx0, didx1, rows0, rows1, acc_sh, sr0, sr1):
    c = lax.axis_index("c")
    s = lax.axis_index("s")
    wid = c * 16 + s
    rpt = NPAD // 16
    pltpu.sync_copy(zeros_hbm.at[pl.ds(s * rpt, rpt)],
                    acc_sh.at[pl.ds(s * rpt, rpt)])
    pltpu.sync_copy(src_hbm.at[wid], src_v)
    plsc.subcore_barrier()

    def body(jj, _):
        j = jj * 2
        pltpu.sync_copy(dst_hbm.at[wid * NCH + j], didx0.at[0])
        pltpu.sync_copy(dst_hbm.at[wid * NCH + j + 1], didx1.at[0])
        cr0 = pltpu.async_copy(u_hbm.at[src_v.at[j]], rows0, sr0)
        cr0.wait()
        cr1 = pltpu.async_copy(u_hbm.at[src_v.at[j + 1]], rows1, sr1)
        pltpu.sync_copy(rows0, acc_sh.at[didx0.at[0]], add=True)
        cr1.wait()
        pltpu.sync_copy(rows1, acc_sh.at[didx1.at[0]], add=True)
        return 0
    lax.fori_loop(0, NCH // 2, body, 0)

    plsc.subcore_barrier()
    pltpu.sync_copy(acc_sh.at[pl.ds(s * rpt, rpt)],
                    out_hbm.at[c].at[pl.ds(s * rpt, rpt)])


@functools.partial(
    pl.kernel, mesh=_mesh,
    out_type=jax.ShapeDtypeStruct((2, G, F), f32),
    scratch_types=[
        pltpu.VMEM((1, CHP), jnp.int32),
        pltpu.VMEM((CHP, F), f32),
        pltpu.VMEM_SHARED((G, F), f32),
        pltpu.SemaphoreType.DMA,
    ])
def _sc_pool(h_hbm, batch_hbm, zg_hbm, out_hbm, bidx_v, rows_v, acc_sh, sem):
    c = lax.axis_index("c")
    s = lax.axis_index("s")
    wid = c * 16 + s
    rpt = G // 16
    pltpu.sync_copy(zg_hbm.at[pl.ds(s * rpt, rpt)],
                    acc_sh.at[pl.ds(s * rpt, rpt)])
    plsc.subcore_barrier()
    for k in range(4):
        j = wid + 32 * k

        @pl.when(j < N // CHP)
        def _():
            pltpu.sync_copy(h_hbm.at[pl.ds(j * CHP, CHP)], rows_v)
            pltpu.sync_copy(batch_hbm.at[j], bidx_v.at[0])
            pltpu.sync_copy(rows_v, acc_sh.at[bidx_v.at[0]], add=True)
    plsc.subcore_barrier()
    pltpu.sync_copy(acc_sh.at[pl.ds(s * rpt, rpt)],
                    out_hbm.at[c].at[pl.ds(s * rpt, rpt)])


def _stats_body(h_ref, out_ref):
    i = pl.program_id(0)

    @pl.when(i == 0)
    def _():
        out_ref[...] = jnp.zeros_like(out_ref)
    h = h_ref[...]
    out_ref[0:1, :] += jnp.sum(h, axis=0, keepdims=True)
    out_ref[1:2, :] += jnp.sum(h * h, axis=0, keepdims=True)


def _stats_call(h):
    return pl.pallas_call(
        _stats_body,
        grid=(NB,),
        in_specs=[pl.BlockSpec((R, F), lambda i: (i, 0))],
        out_specs=pl.BlockSpec((2, F), lambda i: (0, 0)),
        out_shape=jax.ShapeDtypeStruct((2, F), f32),
    )(h)


def _dinv_body(hist_ref, out_ref):
    sall = hist_ref[0] + hist_ref[1]
    deg = sall[:, 0:1] + 1.0
    out_ref[...] = lax.rsqrt(deg)


def _dinv_call(hist):
    return pl.pallas_call(
        _dinv_body,
        out_shape=jax.ShapeDtypeStruct((NPAD, 1), f32),
    )(hist)


def _mm_body(h_ref, W_ref, st_ref, b_ref, d_ref, out_ref, st_out_ref,
             *, relu, use_dinv):
    m = st_ref[0:1, :] / N
    v = st_ref[1:2, :] / N - m * m
    sc = lax.rsqrt(v + EPS)
    hn = (h_ref[...] - m) * sc + 1e-4
    z = jnp.dot(hn, W_ref[...], preferred_element_type=f32) + b_ref[...]
    if relu:
        z = jnp.maximum(z, 0.0)
    if use_dinv:
        z = z * d_ref[...]
    out_ref[...] = z
    i = pl.program_id(0)

    @pl.when(i == 0)
    def _():
        st_out_ref[...] = jnp.zeros_like(st_out_ref)
    st_out_ref[0:1, :] += jnp.sum(z, axis=0, keepdims=True)
    st_out_ref[1:2, :] += jnp.sum(z * z, axis=0, keepdims=True)


def _mm_call(h, W, st, b, dcol, *, relu, use_dinv):
    body = functools.partial(_mm_body, relu=relu, use_dinv=use_dinv)
    return pl.pallas_call(
        body,
        grid=(NB,),
        in_specs=[
            pl.BlockSpec((R, F), lambda i: (i, 0)),
            pl.BlockSpec((F, F), lambda i: (0, 0)),
            pl.BlockSpec((2, F), lambda i: (0, 0)),
            pl.BlockSpec((1, F), lambda i: (0, 0)),
            pl.BlockSpec((R, 1), lambda i: (i, 0)),
        ],
        out_specs=[pl.BlockSpec((R, F), lambda i: (i, 0)),
                   pl.BlockSpec((2, F), lambda i: (0, 0))],
        out_shape=[jax.ShapeDtypeStruct((N, F), f32),
                   jax.ShapeDtypeStruct((2, F), f32)],
    )(h, W, st, b, dcol)


def _post_body(agg_ref, u_ref, d_ref, b_ref, h_ref, st_out_ref):
    a = agg_ref[0] + agg_ref[1]
    h = jnp.maximum(d_ref[...] * (a + u_ref[...]) + b_ref[...], 0.0)
    h_ref[...] = h
    i = pl.program_id(0)

    @pl.when(i == 0)
    def _():
        st_out_ref[...] = jnp.zeros_like(st_out_ref)
    st_out_ref[0:1, :] += jnp.sum(h, axis=0, keepdims=True)
    st_out_ref[1:2, :] += jnp.sum(h * h, axis=0, keepdims=True)


def _post_call(agg, u, dcol, b):
    return pl.pallas_call(
        _post_body,
        grid=(NB,),
        in_specs=[
            pl.BlockSpec((2, R, F), lambda i: (0, i, 0)),
            pl.BlockSpec((R, F), lambda i: (i, 0)),
            pl.BlockSpec((R, 1), lambda i: (i, 0)),
            pl.BlockSpec((1, F), lambda i: (0, 0)),
        ],
        out_specs=[pl.BlockSpec((R, F), lambda i: (i, 0)),
                   pl.BlockSpec((2, F), lambda i: (0, 0))],
        out_shape=[jax.ShapeDtypeStruct((N, F), f32),
                   jax.ShapeDtypeStruct((2, F), f32)],
    )(agg, u, dcol, b)


def _log_softmax(lg):
    mx = jnp.max(lg, axis=1, keepdims=True)
    e = jnp.exp(lg - mx)
    return lg - mx - jnp.log(jnp.sum(e, axis=1, keepdims=True))


def _nodehead_body(h_ref, W1_ref, b1_ref, W2_ref, b2_ref, out_ref):
    t = jnp.maximum(
        jnp.dot(h_ref[...], W1_ref[...], preferred_element_type=f32)
        + b1_ref[...], 0.0)
    lg = jnp.dot(t, W2_ref[...], preferred_element_type=f32) + b2_ref[...]
    out_ref[...] = _log_softmax(lg)


def _nodehead_call(h, W1, b1, W2, b2):
    nc = W2.shape[1]
    return pl.pallas_call(
        _nodehead_body,
        grid=(NB,),
        in_specs=[
            pl.BlockSpec((R, F), lambda i: (i, 0)),
            pl.BlockSpec((F, F), lambda i: (0, 0)),
            pl.BlockSpec((1, F), lambda i: (0, 0)),
            pl.BlockSpec((F, nc), lambda i: (0, 0)),
            pl.BlockSpec((1, nc), lambda i: (0, 0)),
        ],
        out_specs=pl.BlockSpec((R, nc), lambda i: (i, 0)),
        out_shape=jax.ShapeDtypeStruct((N, nc), f32),
    )(h, W1, b1, W2, b2)


def _bn_full(g):
    m = jnp.mean(g, axis=0, keepdims=True)
    v = jnp.mean((g - m) ** 2, axis=0, keepdims=True)
    return (g - m) * lax.rsqrt(v + EPS) + 1e-4


def _graphhead_body(gp_ref, Wl_ref, bl_ref, Wc_ref, bc_ref, out_ref):
    g = gp_ref[0] + gp_ref[1]
    g_ = jnp.maximum(
        jnp.dot(_bn_full(g), Wl_ref[...], preferred_element_type=f32)
        + bl_ref[...], 0.0)
    lg = (jnp.dot(_bn_full(g_), Wc_ref[...], preferred_element_type=f32)
          + bc_ref[...])
    out_ref[...] = _log_softmax(lg)


def _graphhead_call(gp, Wl, bl, Wc, bc):
    nc = Wc.shape[1]
    return pl.pallas_call(
        _graphhead_body,
        out_shape=jax.ShapeDtypeStruct((G, nc), f32),
    )(gp, Wl, bl, Wc, bc)


def kernel(x, edge_index, batch, Wf, bf, W1, b1, W2, b2, W3, b3,
           Wlin, blin, Wcls, bcls, Wn1, bn1, Wn2, bn2):
    src3 = edge_index[0].astype(jnp.int32).reshape(NW, NCH, CH)
    dst3 = edge_index[1].astype(jnp.int32).reshape(NW, NCH, CH)
    srcp = jnp.pad(src3, ((0, 0), (0, 0), (0, CHW - CH)))
    dstp = jnp.pad(dst3, ((0, 0), (0, 0), (0, CHW - CH)),
                   constant_values=TRASH)
    dst2 = dstp.reshape(NW * NCH, CHW)
    batch2 = batch.astype(jnp.int32).reshape(N // CHP, CHP)
    h0 = x[:, 7:]
    zeros_big = jnp.zeros((NPAD, F), f32)
    zeros_g = jnp.zeros((G, F), f32)

    hist = _sc_degree(dstp, jnp.ones((CHW, 16), f32),
                      jnp.zeros((NPAD, 16), f32))
    dinv_col = _dinv_call(hist)

    st = _stats_call(h0)
    h, st = _mm_call(h0, Wf, st, bf.reshape(1, F), dinv_col,
                     relu=True, use_dinv=False)
    for W, b in ((W1, b1), (W2, b2), (W3, b3)):
        u, _ = _mm_call(h, W, st, jnp.zeros((1, F), f32), dinv_col,
                        relu=False, use_dinv=True)
        agg = _sc_prop(u, srcp, dst2, zeros_big)
        h, st = _post_call(agg, u, dinv_col, b.reshape(1, F))

    gp = _sc_pool(h, batch2, zeros_g)
    out_graph = _graphhead_call(gp, Wlin, blin.reshape(1, F),
                                Wcls, bcls.reshape(1, -1))
    out_nodes = _nodehead_call(h, Wn1, bn1.reshape(1, F),
                               Wn2, bn2.reshape(1, -1))
    return out_graph, out_nodes


# speedup vs baseline: 1.0050x; 1.0050x over previous
import functools

import jax
import jax.numpy as jnp
from jax import lax
from jax.experimental import pallas as pl
from jax.experimental.pallas import tpu as pltpu
from jax.experimental.pallas import tpu_sc as plsc

N = 10000
E = 320000
F = 128
G = 128
NPAD = 10240
EPS = 1e-5
NW = 32
EPW = E // NW
CH = 125
CHW = 128
NCH = EPW // CH
SUP = 8
TRASH = NPAD - 1
CHP = 80
R = 1000
NB = N // R
f32 = jnp.float32

_mesh = plsc.VectorSubcoreMesh(core_axis_name="c", subcore_axis_name="s")


@functools.partial(
    pl.kernel, mesh=_mesh,
    out_type=jax.ShapeDtypeStruct((2, NPAD, 16), f32),
    scratch_types=[
        pltpu.VMEM((NCH, CHW), jnp.int32),
        pltpu.VMEM((CHW, 16), f32),
        pltpu.VMEM_SHARED((NPAD, 16), f32),
    ])
def _sc_degree(dst_hbm, ones_hbm, zeros_hbm, out_hbm, dst_v, ones_v, acc_sh):
    c = lax.axis_index("c")
    s = lax.axis_index("s")
    wid = c * 16 + s
    rpt = NPAD // 16
    pltpu.sync_copy(zeros_hbm.at[pl.ds(s * rpt, rpt)],
                    acc_sh.at[pl.ds(s * rpt, rpt)])
    pltpu.sync_copy(dst_hbm.at[wid], dst_v)
    pltpu.sync_copy(ones_hbm, ones_v)
    plsc.subcore_barrier()

    def body(j, _):
        pltpu.sync_copy(ones_v, acc_sh.at[dst_v.at[j]], add=True)
        return 0
    lax.fori_loop(0, NCH, body, 0)

    plsc.subcore_barrier()
    pltpu.sync_copy(acc_sh.at[pl.ds(s * rpt, rpt)],
                    out_hbm.at[c].at[pl.ds(s * rpt, rpt)])


@functools.partial(
    pl.kernel, mesh=_mesh,
    out_type=jax.ShapeDtypeStruct((2, NPAD, F), f32),
    scratch_types=[
        pltpu.VMEM((NCH, CHW), jnp.int32),
        pltpu.VMEM((NCH, CHW), jnp.int32),
        pltpu.VMEM((CHW, F), f32),
        pltpu.VMEM_SHARED((NPAD, F), f32),
        pltpu.SemaphoreType.DMA,
    ])
def _sc_prop(u_hbm, src_hbm, dst_hbm, zeros_hbm, out_hbm,
             src_v, dst_v, rows_v, acc_sh, sem):
    c = lax.axis_index("c")
    s = lax.axis_index("s")
    wid = c * 16 + s
    rpt = NPAD // 16
    pltpu.sync_copy(zeros_hbm.at[pl.ds(s * rpt, rpt)],
                    acc_sh.at[pl.ds(s * rpt, rpt)])
    pltpu.sync_copy(src_hbm.at[wid], src_v)
    pltpu.sync_copy(dst_hbm.at[wid], dst_v)
    plsc.subcore_barrier()

    def body(j, _):
        pltpu.async_copy(u_hbm.at[src_v.at[j]], rows_v, sem).wait()
        pltpu.sync_copy(rows_v, acc_sh.at[dst_v.at[j]], add=True)
        return 0
    lax.fori_loop(0, NCH, body, 0)

    plsc.subcore_barrier()
    pltpu.sync_copy(acc_sh.at[pl.ds(s * rpt, rpt)],
                    out_hbm.at[c].at[pl.ds(s * rpt, rpt)])


@functools.partial(
    pl.kernel, mesh=_mesh,
    out_type=jax.ShapeDtypeStruct((2, G, F), f32),
    scratch_types=[
        pltpu.VMEM((1, CHP), jnp.int32),
        pltpu.VMEM((CHP, F), f32),
        pltpu.VMEM_SHARED((G, F), f32),
        pltpu.SemaphoreType.DMA,
    ])
def _sc_pool(h_hbm, batch_hbm, zg_hbm, out_hbm, bidx_v, rows_v, acc_sh, sem):
    c = lax.axis_index("c")
    s = lax.axis_index("s")
    wid = c * 16 + s
    rpt = G // 16
    pltpu.sync_copy(zg_hbm.at[pl.ds(s * rpt, rpt)],
                    acc_sh.at[pl.ds(s * rpt, rpt)])
    plsc.subcore_barrier()
    for k in range(4):
        j = wid + 32 * k

        @pl.when(j < N // CHP)
        def _():
            pltpu.sync_copy(h_hbm.at[pl.ds(j * CHP, CHP)], rows_v)
            pltpu.sync_copy(batch_hbm.at[j], bidx_v.at[0])
            pltpu.sync_copy(rows_v, acc_sh.at[bidx_v.at[0]], add=True)
    plsc.subcore_barrier()
    pltpu.sync_copy(acc_sh.at[pl.ds(s * rpt, rpt)],
                    out_hbm.at[c].at[pl.ds(s * rpt, rpt)])


def _stats_body(h_ref, out_ref):
    i = pl.program_id(0)

    @pl.when(i == 0)
    def _():
        out_ref[...] = jnp.zeros_like(out_ref)
    h = h_ref[...]
    out_ref[0:1, :] += jnp.sum(h, axis=0, keepdims=True)
    out_ref[1:2, :] += jnp.sum(h * h, axis=0, keepdims=True)


def _stats_call(h):
    return pl.pallas_call(
        _stats_body,
        grid=(NB,),
        in_specs=[pl.BlockSpec((R, F), lambda i: (i, 0))],
        out_specs=pl.BlockSpec((2, F), lambda i: (0, 0)),
        out_shape=jax.ShapeDtypeStruct((2, F), f32),
    )(h)


def _dinv_body(hist_ref, out_ref):
    sall = hist_ref[0] + hist_ref[1]
    deg = sall[:, 0:1] + 1.0
    out_ref[...] = lax.rsqrt(deg)


def _dinv_call(hist):
    return pl.pallas_call(
        _dinv_body,
        out_shape=jax.ShapeDtypeStruct((NPAD, 1), f32),
    )(hist)


def _mm_body(h_ref, W_ref, st_ref, b_ref, d_ref, out_ref, st_out_ref,
             *, relu, use_dinv):
    m = st_ref[0:1, :] / N
    v = st_ref[1:2, :] / N - m * m
    sc = lax.rsqrt(v + EPS)
    hn = (h_ref[...] - m) * sc + 1e-4
    z = jnp.dot(hn, W_ref[...], preferred_element_type=f32) + b_ref[...]
    if relu:
        z = jnp.maximum(z, 0.0)
    if use_dinv:
        z = z * d_ref[...]
    out_ref[...] = z
    i = pl.program_id(0)

    @pl.when(i == 0)
    def _():
        st_out_ref[...] = jnp.zeros_like(st_out_ref)
    st_out_ref[0:1, :] += jnp.sum(z, axis=0, keepdims=True)
    st_out_ref[1:2, :] += jnp.sum(z * z, axis=0, keepdims=True)


def _mm_call(h, W, st, b, dcol, *, relu, use_dinv):
    body = functools.partial(_mm_body, relu=relu, use_dinv=use_dinv)
    return pl.pallas_call(
        body,
        grid=(NB,),
        in_specs=[
            pl.BlockSpec((R, F), lambda i: (i, 0)),
            pl.BlockSpec((F, F), lambda i: (0, 0)),
            pl.BlockSpec((2, F), lambda i: (0, 0)),
            pl.BlockSpec((1, F), lambda i: (0, 0)),
            pl.BlockSpec((R, 1), lambda i: (i, 0)),
        ],
        out_specs=[pl.BlockSpec((R, F), lambda i: (i, 0)),
                   pl.BlockSpec((2, F), lambda i: (0, 0))],
        out_shape=[jax.ShapeDtypeStruct((N, F), f32),
                   jax.ShapeDtypeStruct((2, F), f32)],
    )(h, W, st, b, dcol)


def _post_body(agg_ref, u_ref, d_ref, b_ref, h_ref, st_out_ref):
    a = agg_ref[0] + agg_ref[1]
    h = jnp.maximum(d_ref[...] * (a + u_ref[...]) + b_ref[...], 0.0)
    h_ref[...] = h
    i = pl.program_id(0)

    @pl.when(i == 0)
    def _():
        st_out_ref[...] = jnp.zeros_like(st_out_ref)
    st_out_ref[0:1, :] += jnp.sum(h, axis=0, keepdims=True)
    st_out_ref[1:2, :] += jnp.sum(h * h, axis=0, keepdims=True)


def _post_call(agg, u, dcol, b):
    return pl.pallas_call(
        _post_body,
        grid=(NB,),
        in_specs=[
            pl.BlockSpec((2, R, F), lambda i: (0, i, 0)),
            pl.BlockSpec((R, F), lambda i: (i, 0)),
            pl.BlockSpec((R, 1), lambda i: (i, 0)),
            pl.BlockSpec((1, F), lambda i: (0, 0)),
        ],
        out_specs=[pl.BlockSpec((R, F), lambda i: (i, 0)),
                   pl.BlockSpec((2, F), lambda i: (0, 0))],
        out_shape=[jax.ShapeDtypeStruct((N, F), f32),
                   jax.ShapeDtypeStruct((2, F), f32)],
    )(agg, u, dcol, b)


def _log_softmax(lg):
    mx = jnp.max(lg, axis=1, keepdims=True)
    e = jnp.exp(lg - mx)
    return lg - mx - jnp.log(jnp.sum(e, axis=1, keepdims=True))


def _nodehead_body(h_ref, W1_ref, b1_ref, W2_ref, b2_ref, out_ref):
    t = jnp.maximum(
        jnp.dot(h_ref[...], W1_ref[...], preferred_element_type=f32)
        + b1_ref[...], 0.0)
    lg = jnp.dot(t, W2_ref[...], preferred_element_type=f32) + b2_ref[...]
    out_ref[...] = _log_softmax(lg)


def _nodehead_call(h, W1, b1, W2, b2):
    nc = W2.shape[1]
    return pl.pallas_call(
        _nodehead_body,
        grid=(NB,),
        in_specs=[
            pl.BlockSpec((R, F), lambda i: (i, 0)),
            pl.BlockSpec((F, F), lambda i: (0, 0)),
            pl.BlockSpec((1, F), lambda i: (0, 0)),
            pl.BlockSpec((F, nc), lambda i: (0, 0)),
            pl.BlockSpec((1, nc), lambda i: (0, 0)),
        ],
        out_specs=pl.BlockSpec((R, nc), lambda i: (i, 0)),
        out_shape=jax.ShapeDtypeStruct((N, nc), f32),
    )(h, W1, b1, W2, b2)


def _bn_full(g):
    m = jnp.mean(g, axis=0, keepdims=True)
    v = jnp.mean((g - m) ** 2, axis=0, keepdims=True)
    return (g - m) * lax.rsqrt(v + EPS) + 1e-4


def _graphhead_body(gp_ref, Wl_ref, bl_ref, Wc_ref, bc_ref, out_ref):
    g = gp_ref[0] + gp_ref[1]
    g_ = jnp.maximum(
        jnp.dot(_bn_full(g), Wl_ref[...], preferred_element_type=f32)
        + bl_ref[...], 0.0)
    lg = (jnp.dot(_bn_full(g_), Wc_ref[...], preferred_element_type=f32)
          + bc_ref[...])
    out_ref[...] = _log_softmax(lg)


def _graphhead_call(gp, Wl, bl, Wc, bc):
    nc = Wc.shape[1]
    return pl.pallas_call(
        _graphhead_body,
        out_shape=jax.ShapeDtypeStruct((G, nc), f32),
    )(gp, Wl, bl, Wc, bc)


def kernel(x, edge_index, batch, Wf, bf, W1, b1, W2, b2, W3, b3,
           Wlin, blin, Wcls, bcls, Wn1, bn1, Wn2, bn2):
    src3 = edge_index[0].astype(jnp.int32).reshape(NW, NCH, CH)
    dst3 = edge_index[1].astype(jnp.int32).reshape(NW, NCH, CH)
    srcp = jnp.pad(src3, ((0, 0), (0, 0), (0, CHW - CH)))
    dstp = jnp.pad(dst3, ((0, 0), (0, 0), (0, CHW - CH)),
                   constant_values=TRASH)
    batch2 = batch.astype(jnp.int32).reshape(N // CHP, CHP)
    h0 = x[:, 7:]
    zeros_big = jnp.zeros((NPAD, F), f32)
    zeros_g = jnp.zeros((G, F), f32)

    hist = _sc_degree(dstp, jnp.ones((CHW, 16), f32),
                      jnp.zeros((NPAD, 16), f32))
    dinv_col = _dinv_call(hist)

    st = _stats_call(h0)
    h, st = _mm_call(h0, Wf, st, bf.reshape(1, F), dinv_col,
                     relu=True, use_dinv=False)
    for W, b in ((W1, b1), (W2, b2), (W3, b3)):
        u, _ = _mm_call(h, W, st, jnp.zeros((1, F), f32), dinv_col,
                        relu=False, use_dinv=True)
        agg = _sc_prop(u, srcp, dstp, zeros_big)
        h, st = _post_call(agg, u, dinv_col, b.reshape(1, F))

    gp = _sc_pool(h, batch2, zeros_g)
    out_graph = _graphhead_call(gp, Wlin, blin.reshape(1, F),
                                Wcls, bcls.reshape(1, -1))
    out_nodes = _nodehead_call(h, Wn1, bn1.reshape(1, F),
                               Wn2, bn2.reshape(1, -1))
    return out_graph, out_nodes


# speedup vs baseline: 1.8469x; 1.8378x over previous
import functools

import jax
import jax.numpy as jnp
from jax import lax
from jax.experimental import pallas as pl
from jax.experimental.pallas import tpu as pltpu
from jax.experimental.pallas import tpu_sc as plsc

N = 10000
E = 320000
F = 128
G = 128
NPAD = 10240
EPS = 1e-5
NW = 32
EPW = E // NW
CH = 125
CHW = 128
NCH = EPW // CH
SUP = 8
TRASH = NPAD - 1
CHP = 80
R = 1000
NB = N // R
f32 = jnp.float32

_mesh = plsc.VectorSubcoreMesh(core_axis_name="c", subcore_axis_name="s")


@functools.partial(
    pl.kernel, mesh=_mesh,
    out_type=jax.ShapeDtypeStruct((2, NPAD, 16), f32),
    scratch_types=[
        pltpu.VMEM((NCH, CHW), jnp.int32),
        pltpu.VMEM((CHW, 16), f32),
        pltpu.VMEM_SHARED((NPAD, 16), f32),
    ])
def _sc_degree(dst_hbm, ones_hbm, zeros_hbm, out_hbm, dst_v, ones_v, acc_sh):
    c = lax.axis_index("c")
    s = lax.axis_index("s")
    wid = c * 16 + s
    rpt = NPAD // 16
    pltpu.sync_copy(zeros_hbm.at[pl.ds(s * rpt, rpt)],
                    acc_sh.at[pl.ds(s * rpt, rpt)])
    pltpu.sync_copy(dst_hbm.at[wid], dst_v)
    pltpu.sync_copy(ones_hbm, ones_v)
    plsc.subcore_barrier()

    def body(j, _):
        pltpu.sync_copy(ones_v, acc_sh.at[dst_v.at[j]], add=True)
        return 0
    lax.fori_loop(0, NCH, body, 0)

    plsc.subcore_barrier()
    pltpu.sync_copy(acc_sh.at[pl.ds(s * rpt, rpt)],
                    out_hbm.at[c].at[pl.ds(s * rpt, rpt)])


@functools.partial(
    pl.kernel, mesh=_mesh,
    out_type=jax.ShapeDtypeStruct((2, NPAD, F), f32),
    scratch_types=[
        pltpu.VMEM((NCH, CH), jnp.int32),
        pltpu.VMEM((NCH, CH), jnp.int32),
        pltpu.VMEM((CH, F), f32),
        pltpu.VMEM_SHARED((NPAD, F), f32),
        pltpu.SemaphoreType.DMA,
    ])
def _sc_prop(u_hbm, src_hbm, dst_hbm, zeros_hbm, out_hbm,
             src_v, dst_v, rows_v, acc_sh, sem):
    c = lax.axis_index("c")
    s = lax.axis_index("s")
    wid = c * 16 + s
    rpt = NPAD // 16
    pltpu.sync_copy(zeros_hbm.at[pl.ds(s * rpt, rpt)],
                    acc_sh.at[pl.ds(s * rpt, rpt)])
    pltpu.sync_copy(src_hbm.at[wid], src_v)
    pltpu.sync_copy(dst_hbm.at[wid], dst_v)
    plsc.subcore_barrier()

    def body(j, _):
        pltpu.async_copy(u_hbm.at[src_v.at[j]], rows_v, sem).wait()
        pltpu.sync_copy(rows_v, acc_sh.at[dst_v.at[j]], add=True)
        return 0
    lax.fori_loop(0, NCH, body, 0)

    plsc.subcore_barrier()
    pltpu.sync_copy(acc_sh.at[pl.ds(s * rpt, rpt)],
                    out_hbm.at[c].at[pl.ds(s * rpt, rpt)])


@functools.partial(
    pl.kernel, mesh=_mesh,
    out_type=jax.ShapeDtypeStruct((2, G, F), f32),
    scratch_types=[
        pltpu.VMEM((1, CHP), jnp.int32),
        pltpu.VMEM((CHP, F), f32),
        pltpu.VMEM_SHARED((G, F), f32),
        pltpu.SemaphoreType.DMA,
    ])
def _sc_pool(h_hbm, batch_hbm, zg_hbm, out_hbm, bidx_v, rows_v, acc_sh, sem):
    c = lax.axis_index("c")
    s = lax.axis_index("s")
    wid = c * 16 + s
    rpt = G // 16
    pltpu.sync_copy(zg_hbm.at[pl.ds(s * rpt, rpt)],
                    acc_sh.at[pl.ds(s * rpt, rpt)])
    plsc.subcore_barrier()
    for k in range(4):
        j = wid + 32 * k

        @pl.when(j < N // CHP)
        def _():
            pltpu.sync_copy(h_hbm.at[pl.ds(j * CHP, CHP)], rows_v)
            pltpu.sync_copy(batch_hbm.at[j], bidx_v.at[0])
            pltpu.sync_copy(rows_v, acc_sh.at[bidx_v.at[0]], add=True)
    plsc.subcore_barrier()
    pltpu.sync_copy(acc_sh.at[pl.ds(s * rpt, rpt)],
                    out_hbm.at[c].at[pl.ds(s * rpt, rpt)])


def _stats_body(h_ref, out_ref):
    i = pl.program_id(0)

    @pl.when(i == 0)
    def _():
        out_ref[...] = jnp.zeros_like(out_ref)
    h = h_ref[...]
    out_ref[0:1, :] += jnp.sum(h, axis=0, keepdims=True)
    out_ref[1:2, :] += jnp.sum(h * h, axis=0, keepdims=True)


def _stats_call(h):
    return pl.pallas_call(
        _stats_body,
        grid=(NB,),
        in_specs=[pl.BlockSpec((R, F), lambda i: (i, 0))],
        out_specs=pl.BlockSpec((2, F), lambda i: (0, 0)),
        out_shape=jax.ShapeDtypeStruct((2, F), f32),
    )(h)


def _dinv_body(hist_ref, out_ref):
    sall = hist_ref[0] + hist_ref[1]
    deg = sall[:, 0:1] + 1.0
    out_ref[...] = lax.rsqrt(deg)


def _dinv_call(hist):
    return pl.pallas_call(
        _dinv_body,
        out_shape=jax.ShapeDtypeStruct((NPAD, 1), f32),
    )(hist)


def _mm_body(h_ref, W_ref, st_ref, b_ref, d_ref, out_ref, st_out_ref,
             *, relu, use_dinv):
    m = st_ref[0:1, :] / N
    v = st_ref[1:2, :] / N - m * m
    sc = lax.rsqrt(v + EPS)
    hn = (h_ref[...] - m) * sc + 1e-4
    z = jnp.dot(hn, W_ref[...], preferred_element_type=f32) + b_ref[...]
    if relu:
        z = jnp.maximum(z, 0.0)
    if use_dinv:
        z = z * d_ref[...]
    out_ref[...] = z
    i = pl.program_id(0)

    @pl.when(i == 0)
    def _():
        st_out_ref[...] = jnp.zeros_like(st_out_ref)
    st_out_ref[0:1, :] += jnp.sum(z, axis=0, keepdims=True)
    st_out_ref[1:2, :] += jnp.sum(z * z, axis=0, keepdims=True)


def _mm_call(h, W, st, b, dcol, *, relu, use_dinv):
    body = functools.partial(_mm_body, relu=relu, use_dinv=use_dinv)
    return pl.pallas_call(
        body,
        grid=(NB,),
        in_specs=[
            pl.BlockSpec((R, F), lambda i: (i, 0)),
            pl.BlockSpec((F, F), lambda i: (0, 0)),
            pl.BlockSpec((2, F), lambda i: (0, 0)),
            pl.BlockSpec((1, F), lambda i: (0, 0)),
            pl.BlockSpec((R, 1), lambda i: (i, 0)),
        ],
        out_specs=[pl.BlockSpec((R, F), lambda i: (i, 0)),
                   pl.BlockSpec((2, F), lambda i: (0, 0))],
        out_shape=[jax.ShapeDtypeStruct((N, F), f32),
                   jax.ShapeDtypeStruct((2, F), f32)],
    )(h, W, st, b, dcol)


def _post_body(agg_ref, u_ref, d_ref, b_ref, h_ref, st_out_ref):
    a = agg_ref[0] + agg_ref[1]
    h = jnp.maximum(d_ref[...] * (a + u_ref[...]) + b_ref[...], 0.0)
    h_ref[...] = h
    i = pl.program_id(0)

    @pl.when(i == 0)
    def _():
        st_out_ref[...] = jnp.zeros_like(st_out_ref)
    st_out_ref[0:1, :] += jnp.sum(h, axis=0, keepdims=True)
    st_out_ref[1:2, :] += jnp.sum(h * h, axis=0, keepdims=True)


def _post_call(agg, u, dcol, b):
    return pl.pallas_call(
        _post_body,
        grid=(NB,),
        in_specs=[
            pl.BlockSpec((2, R, F), lambda i: (0, i, 0)),
            pl.BlockSpec((R, F), lambda i: (i, 0)),
            pl.BlockSpec((R, 1), lambda i: (i, 0)),
            pl.BlockSpec((1, F), lambda i: (0, 0)),
        ],
        out_specs=[pl.BlockSpec((R, F), lambda i: (i, 0)),
                   pl.BlockSpec((2, F), lambda i: (0, 0))],
        out_shape=[jax.ShapeDtypeStruct((N, F), f32),
                   jax.ShapeDtypeStruct((2, F), f32)],
    )(agg, u, dcol, b)


def _log_softmax(lg):
    mx = jnp.max(lg, axis=1, keepdims=True)
    e = jnp.exp(lg - mx)
    return lg - mx - jnp.log(jnp.sum(e, axis=1, keepdims=True))


def _nodehead_body(h_ref, W1_ref, b1_ref, W2_ref, b2_ref, out_ref):
    t = jnp.maximum(
        jnp.dot(h_ref[...], W1_ref[...], preferred_element_type=f32)
        + b1_ref[...], 0.0)
    lg = jnp.dot(t, W2_ref[...], preferred_element_type=f32) + b2_ref[...]
    out_ref[...] = _log_softmax(lg)


def _nodehead_call(h, W1, b1, W2, b2):
    nc = W2.shape[1]
    return pl.pallas_call(
        _nodehead_body,
        grid=(NB,),
        in_specs=[
            pl.BlockSpec((R, F), lambda i: (i, 0)),
            pl.BlockSpec((F, F), lambda i: (0, 0)),
            pl.BlockSpec((1, F), lambda i: (0, 0)),
            pl.BlockSpec((F, nc), lambda i: (0, 0)),
            pl.BlockSpec((1, nc), lambda i: (0, 0)),
        ],
        out_specs=pl.BlockSpec((R, nc), lambda i: (i, 0)),
        out_shape=jax.ShapeDtypeStruct((N, nc), f32),
    )(h, W1, b1, W2, b2)


def _bn_full(g):
    m = jnp.mean(g, axis=0, keepdims=True)
    v = jnp.mean((g - m) ** 2, axis=0, keepdims=True)
    return (g - m) * lax.rsqrt(v + EPS) + 1e-4


def _graphhead_body(gp_ref, Wl_ref, bl_ref, Wc_ref, bc_ref, out_ref):
    g = gp_ref[0] + gp_ref[1]
    g_ = jnp.maximum(
        jnp.dot(_bn_full(g), Wl_ref[...], preferred_element_type=f32)
        + bl_ref[...], 0.0)
    lg = (jnp.dot(_bn_full(g_), Wc_ref[...], preferred_element_type=f32)
          + bc_ref[...])
    out_ref[...] = _log_softmax(lg)


def _graphhead_call(gp, Wl, bl, Wc, bc):
    nc = Wc.shape[1]
    return pl.pallas_call(
        _graphhead_body,
        out_shape=jax.ShapeDtypeStruct((G, nc), f32),
    )(gp, Wl, bl, Wc, bc)


def kernel(x, edge_index, batch, Wf, bf, W1, b1, W2, b2, W3, b3,
           Wlin, blin, Wcls, bcls, Wn1, bn1, Wn2, bn2):
    src3 = edge_index[0].astype(jnp.int32).reshape(NW, NCH, CH)
    dst3 = edge_index[1].astype(jnp.int32).reshape(NW, NCH, CH)
    dstp = jnp.pad(dst3, ((0, 0), (0, 0), (0, CHW - CH)),
                   constant_values=TRASH)
    batch2 = batch.astype(jnp.int32).reshape(N // CHP, CHP)
    h0 = x[:, 7:]
    zeros_big = jnp.zeros((NPAD, F), f32)
    zeros_g = jnp.zeros((G, F), f32)

    hist = _sc_degree(dstp, jnp.ones((CHW, 16), f32),
                      jnp.zeros((NPAD, 16), f32))
    dinv_col = _dinv_call(hist)

    st = _stats_call(h0)
    h, st = _mm_call(h0, Wf, st, bf.reshape(1, F), dinv_col,
                     relu=True, use_dinv=False)
    for W, b in ((W1, b1), (W2, b2), (W3, b3)):
        u, _ = _mm_call(h, W, st, jnp.zeros((1, F), f32), dinv_col,
                        relu=False, use_dinv=True)
        agg = _sc_prop(u, src3, dst3, zeros_big)
        h, st = _post_call(agg, u, dinv_col, b.reshape(1, F))

    gp = _sc_pool(h, batch2, zeros_g)
    out_graph = _graphhead_call(gp, Wlin, blin.reshape(1, F),
                                Wcls, bcls.reshape(1, -1))
    out_nodes = _nodehead_call(h, Wn1, bn1.reshape(1, F),
                               Wn2, bn2.reshape(1, -1))
    return out_graph, out_nodes
